# baseline (speedup 1.0000x reference)
import functools

import jax
import jax.numpy as jnp
from jax import lax
from jax.experimental import pallas as pl
from jax.experimental.pallas import tpu as pltpu
from jax.experimental.pallas import tpu_sc as plsc

B = 1024
H, W = 21, 79
HW = H * W
GPAD = 1664
CHAR_DIM = 96
COLOR_DIM = 16
NBINS = CHAR_DIM * COLOR_DIM
PBINS = 1664
NCHUNK = NBINS // 16
SENT = 2047
MAX_LEN = 64
PAD_CHAR = 128
PAD_COLOR = COLOR_DIM
HIDDEN = 32
GROUP = 8


def _sc_bag_body(chars_hbm, colors_hbm, keys_hbm, chars_v, colors_v, pres, outbuf):
    info = plsc.get_sparse_core_info()
    nc = info.num_cores
    nw = nc * info.num_subcores
    spw = B // nw
    wid = lax.axis_index("s") * nc + lax.axis_index("c")

    zero16 = jnp.zeros((16,), jnp.int32)
    sent16 = jnp.full((16,), SENT, jnp.int32)
    iota16 = lax.iota(jnp.int32, 16)

    def _zero(i, _):
        pres[pl.ds(i * 16, 16)] = zero16
        return 0

    lax.fori_loop(0, PBINS // 16, _zero, 0)

    def _group(g, _):
        base = wid * spw + g * GROUP
        pltpu.sync_copy(chars_hbm.at[pl.ds(base, GROUP)], chars_v)
        pltpu.sync_copy(colors_hbm.at[pl.ds(base, GROUP)], colors_v)

        def _sample(jj, _):
            s = base + jj
            marker = jnp.full((16,), s + 1, jnp.int32)

            def _scatter(i, _):
                c = chars_v[jj, pl.ds(i * 16, 16)]
                l = colors_v[jj, pl.ds(i * 16, 16)]
                plsc.store_scatter(pres, [c * 16 + l], marker)
                return 0

            lax.fori_loop(0, GPAD // 16, _scatter, 0)

            for cc in range(5):
                outbuf[pl.ds(cc * 16, 16)] = sent16

            def _cond(carry):
                i, off = carry
                return jnp.logical_and(i < NCHUNK, off < MAX_LEN)

            def _compact(carry):
                i, off = carry
                m = pres[pl.ds(i * 16, 16)] == marker
                mi = m.astype(jnp.int32)
                pos = plsc.cumsum(mi)
                plsc.store_scatter(outbuf, [off + pos - 1], iota16 + i * 16, mask=m)
                return i + 1, off + jnp.sum(mi)

            lax.while_loop(_cond, _compact, (jnp.int32(0), jnp.int32(0)))
            pltpu.sync_copy(outbuf.at[pl.ds(0, MAX_LEN)], keys_hbm.at[s])
            return 0

        lax.fori_loop(0, GROUP, _sample, 0)
        return 0

    lax.fori_loop(0, spw // GROUP, _group, 0)


@functools.cache
def _make_sc_bag():
    return pl.kernel(
        _sc_bag_body,
        mesh=plsc.VectorSubcoreMesh(core_axis_name="c", subcore_axis_name="s"),
        compiler_params=pltpu.CompilerParams(
            needs_layout_passes=False, use_tc_tiling_on_sc=False),
        out_type=jax.ShapeDtypeStruct((B, MAX_LEN), jnp.int32),
        scratch_types=[
            pltpu.VMEM((GROUP, GPAD), jnp.int32),
            pltpu.VMEM((GROUP, GPAD), jnp.int32),
            pltpu.VMEM((PBINS,), jnp.int32),
            pltpu.VMEM((80,), jnp.int32),
        ],
    )


def _tc_body(keys_ref, ct_ref, lt_ref, wih_ref, whh_ref, bih_ref, bhh_ref,
             feat_ref, emb_ref, cbag_ref, lbag_ref):
    ct = ct_ref[...]
    lt = lt_ref[...]
    keys = keys_ref[...]
    valid = keys < NBINS

    cbag_ref[...] = jnp.where(valid, keys >> 4, PAD_CHAR)
    lbag_ref[...] = jnp.where(valid, keys & 15, PAD_COLOR)

    ci = jnp.where(valid, keys >> 4, CHAR_DIM)
    lj = jnp.where(valid, keys & 15, COLOR_DIM) + (CHAR_DIM + 1)
    iota_j = lax.broadcasted_iota(jnp.int32, (1, 114), 1)

    wih = wih_ref[...]
    whh = whh_ref[...]
    bias = bih_ref[...] + bhh_ref[...]

    z97x4 = jnp.zeros((CHAR_DIM + 1, 4), jnp.float32)
    z17x16 = jnp.zeros((COLOR_DIM + 1, 16), jnp.float32)
    te = jnp.concatenate([
        jnp.concatenate([ct, z97x4], axis=1),
        jnp.concatenate([z17x16, lt], axis=1)], axis=0)
    tw = lax.dot_general(te, wih, (((1,), (1,)), ((), ())),
                         preferred_element_type=jnp.float32)
    tt = jnp.concatenate([te, tw], axis=1)
    tt_hi = tt.astype(jnp.bfloat16)
    tt_lo = (tt - tt_hi.astype(jnp.float32)).astype(jnp.bfloat16)

    h = jnp.zeros((B, HIDDEN), jnp.float32)
    for t in range(MAX_LEN):
        oh = ((ci[:, t:t + 1] == iota_j) | (lj[:, t:t + 1] == iota_j)
              ).astype(jnp.bfloat16)
        z = (lax.dot_general(oh, tt_hi, (((1,), (0,)), ((), ())),
                             preferred_element_type=jnp.float32)
             + lax.dot_general(oh, tt_lo, (((1,), (0,)), ((), ())),
                               preferred_element_type=jnp.float32))
        emb_ref[:, t, :] = z[:, :20]
        hn = jnp.tanh(
            z[:, 20:]
            + lax.dot_general(h, whh, (((1,), (1,)), ((), ())),
                              preferred_element_type=jnp.float32)
            + bias)
        h = jnp.where(valid[:, t:t + 1], hn, h)
    feat_ref[...] = h


def _tc_embed_rnn(keys, char_table, color_table, W_ih, W_hh, b_ih, b_hh):
    return pl.pallas_call(
        _tc_body,
        out_shape=(
            jax.ShapeDtypeStruct((B, HIDDEN), jnp.float32),
            jax.ShapeDtypeStruct((B, MAX_LEN, 20), jnp.float32),
            jax.ShapeDtypeStruct((B, MAX_LEN), jnp.int32),
            jax.ShapeDtypeStruct((B, MAX_LEN), jnp.int32),
        ),
    )(keys, char_table, color_table, W_ih, W_hh,
      b_ih.reshape(1, HIDDEN), b_hh.reshape(1, HIDDEN))


def kernel(glyph_chars, glyph_colors, char_table, color_table, W_ih, W_hh, b_ih, b_hh):
    chars2 = glyph_chars.reshape(B, HW).astype(jnp.int32)
    colors2 = glyph_colors.reshape(B, HW).astype(jnp.int32)
    chars2 = jnp.pad(chars2, ((0, 0), (0, GPAD - HW)), constant_values=CHAR_DIM)
    colors2 = jnp.pad(colors2, ((0, 0), (0, GPAD - HW)), constant_values=0)

    keys = _make_sc_bag()(chars2, colors2)
    features, emb, cbag, lbag = _tc_embed_rnn(
        keys, char_table, color_table, W_ih, W_hh, b_ih, b_hh)
    bag = jnp.stack([cbag, lbag], axis=-1)
    return features, emb, bag


# speedup vs baseline: 1.1123x; 1.1123x over previous
import functools

import jax
import jax.numpy as jnp
from jax import lax
from jax.experimental import pallas as pl
from jax.experimental.pallas import tpu as pltpu
from jax.experimental.pallas import tpu_sc as plsc

B = 1024
H, W = 21, 79
HW = H * W
GPAD = 1664
CHAR_DIM = 96
COLOR_DIM = 16
NBINS = CHAR_DIM * COLOR_DIM
PBINS = 1664
NCHUNK = NBINS // 16
SENT = 2047
MAX_LEN = 64
PAD_CHAR = 128
PAD_COLOR = COLOR_DIM
HIDDEN = 32
GROUP = 8


def _sc_bag_body(chars_hbm, colors_hbm, keys_hbm, chars_v, colors_v, pres, outbuf):
    info = plsc.get_sparse_core_info()
    nc = info.num_cores
    nw = nc * info.num_subcores
    spw = B // nw
    wid = lax.axis_index("s") * nc + lax.axis_index("c")

    zero16 = jnp.zeros((16,), jnp.int32)
    sent16 = jnp.full((16,), SENT, jnp.int32)
    iota16 = lax.iota(jnp.int32, 16)

    def _zero(i, _):
        pres[pl.ds(i * 16, 16)] = zero16
        return 0

    lax.fori_loop(0, PBINS // 16, _zero, 0)

    def _group(g, _):
        base = wid * spw + g * GROUP
        pltpu.sync_copy(chars_hbm.at[pl.ds(base, GROUP)], chars_v)
        pltpu.sync_copy(colors_hbm.at[pl.ds(base, GROUP)], colors_v)

        def _sample(jj, _):
            s = base + jj
            marker = jnp.full((16,), s + 1, jnp.int32)

            def _scatter(i, _):
                c = chars_v[jj, pl.ds(i * 16, 16)]
                l = colors_v[jj, pl.ds(i * 16, 16)]
                plsc.store_scatter(pres, [c * 16 + l], marker)
                return 0

            lax.fori_loop(0, GPAD // 16, _scatter, 0)

            for cc in range(5):
                outbuf[pl.ds(cc * 16, 16)] = sent16

            def _cond(carry):
                i, off = carry
                return jnp.logical_and(i < NCHUNK, off < MAX_LEN)

            def _compact(carry):
                i, off = carry
                m = pres[pl.ds(i * 16, 16)] == marker
                mi = m.astype(jnp.int32)
                pos = plsc.cumsum(mi)
                plsc.store_scatter(outbuf, [off + pos - 1], iota16 + i * 16, mask=m)
                return i + 1, off + jnp.sum(mi)

            lax.while_loop(_cond, _compact, (jnp.int32(0), jnp.int32(0)))
            pltpu.sync_copy(outbuf.at[pl.ds(0, MAX_LEN)], keys_hbm.at[s])
            return 0

        lax.fori_loop(0, GROUP, _sample, 0)
        return 0

    lax.fori_loop(0, spw // GROUP, _group, 0)


@functools.cache
def _make_sc_bag():
    return pl.kernel(
        _sc_bag_body,
        mesh=plsc.VectorSubcoreMesh(core_axis_name="c", subcore_axis_name="s"),
        compiler_params=pltpu.CompilerParams(
            needs_layout_passes=False, use_tc_tiling_on_sc=False),
        out_type=jax.ShapeDtypeStruct((B, MAX_LEN), jnp.int32),
        scratch_types=[
            pltpu.VMEM((GROUP, GPAD), jnp.int32),
            pltpu.VMEM((GROUP, GPAD), jnp.int32),
            pltpu.VMEM((PBINS,), jnp.int32),
            pltpu.VMEM((80,), jnp.int32),
        ],
    )


def _tc_body(keys_ref, ct_ref, lt_ref, wih_ref, whh_ref, bih_ref, bhh_ref,
             feat_ref, emb_ref, cbag_ref, lbag_ref):
    ct = ct_ref[...]
    lt = lt_ref[...]
    keys = keys_ref[...]
    valid = keys < NBINS

    cbag_ref[...] = jnp.where(valid, keys >> 4, PAD_CHAR)
    lbag_ref[...] = jnp.where(valid, keys & 15, PAD_COLOR)

    ci = jnp.where(valid, keys >> 4, CHAR_DIM)
    lj = jnp.where(valid, keys & 15, COLOR_DIM) + (CHAR_DIM + 1)
    iota_j = lax.broadcasted_iota(jnp.int32, (1, 114), 1)

    wih = wih_ref[...]
    whh = whh_ref[...]
    bias = bih_ref[...] + bhh_ref[...]

    z97x4 = jnp.zeros((CHAR_DIM + 1, 4), jnp.float32)
    z17x16 = jnp.zeros((COLOR_DIM + 1, 16), jnp.float32)
    te = jnp.concatenate([
        jnp.concatenate([ct, z97x4], axis=1),
        jnp.concatenate([z17x16, lt], axis=1)], axis=0)
    tw = lax.dot_general(te, wih, (((1,), (1,)), ((), ())),
                         preferred_element_type=jnp.float32)
    tt = jnp.concatenate([te, tw], axis=1)
    tt_hi = tt.astype(jnp.bfloat16)
    tt_lo = (tt - tt_hi.astype(jnp.float32)).astype(jnp.bfloat16)

    h = jnp.zeros((B, HIDDEN), jnp.float32)
    for t in range(MAX_LEN):
        oh = ((ci[:, t:t + 1] == iota_j) | (lj[:, t:t + 1] == iota_j)
              ).astype(jnp.bfloat16)
        z = (lax.dot_general(oh, tt_hi, (((1,), (0,)), ((), ())),
                             preferred_element_type=jnp.float32)
             + lax.dot_general(oh, tt_lo, (((1,), (0,)), ((), ())),
                               preferred_element_type=jnp.float32))
        emb_ref[:, t * 20:(t + 1) * 20] = z[:, :20]
        hn = jnp.tanh(
            z[:, 20:]
            + lax.dot_general(h, whh, (((1,), (1,)), ((), ())),
                              preferred_element_type=jnp.float32)
            + bias)
        h = jnp.where(valid[:, t:t + 1], hn, h)
    feat_ref[...] = h


def _tc_embed_rnn(keys, char_table, color_table, W_ih, W_hh, b_ih, b_hh):
    return pl.pallas_call(
        _tc_body,
        out_shape=(
            jax.ShapeDtypeStruct((B, HIDDEN), jnp.float32),
            jax.ShapeDtypeStruct((B, MAX_LEN * 20), jnp.float32),
            jax.ShapeDtypeStruct((B, MAX_LEN), jnp.int32),
            jax.ShapeDtypeStruct((B, MAX_LEN), jnp.int32),
        ),
    )(keys, char_table, color_table, W_ih, W_hh,
      b_ih.reshape(1, HIDDEN), b_hh.reshape(1, HIDDEN))


def kernel(glyph_chars, glyph_colors, char_table, color_table, W_ih, W_hh, b_ih, b_hh):
    chars2 = glyph_chars.reshape(B, HW).astype(jnp.int32)
    colors2 = glyph_colors.reshape(B, HW).astype(jnp.int32)
    chars2 = jnp.pad(chars2, ((0, 0), (0, GPAD - HW)), constant_values=CHAR_DIM)
    colors2 = jnp.pad(colors2, ((0, 0), (0, GPAD - HW)), constant_values=0)

    keys = _make_sc_bag()(chars2, colors2)
    features, emb_flat, cbag, lbag = _tc_embed_rnn(
        keys, char_table, color_table, W_ih, W_hh, b_ih, b_hh)
    bag = jnp.stack([cbag, lbag], axis=-1)
    return features, emb_flat.reshape(B, MAX_LEN, 20), bag


# speedup vs baseline: 1.1362x; 1.0215x over previous
import functools

import jax
import jax.numpy as jnp
from jax import lax
from jax.experimental import pallas as pl
from jax.experimental.pallas import tpu as pltpu
from jax.experimental.pallas import tpu_sc as plsc

B = 1024
H, W = 21, 79
HW = H * W
GPAD = 1664
CHAR_DIM = 96
COLOR_DIM = 16
NBINS = CHAR_DIM * COLOR_DIM
PBINS = 1664
NCHUNK = NBINS // 16
SENT = 2047
MAX_LEN = 64
PAD_CHAR = 128
PAD_COLOR = COLOR_DIM
HIDDEN = 32
GROUP = 8


def _sc_bag_body(chars_hbm, colors_hbm, keys_hbm, chars_v, colors_v, pres, outbuf):
    info = plsc.get_sparse_core_info()
    nc = info.num_cores
    nw = nc * info.num_subcores
    spw = B // nw
    wid = lax.axis_index("s") * nc + lax.axis_index("c")

    zero16 = jnp.zeros((16,), jnp.int32)
    sent16 = jnp.full((16,), SENT, jnp.int32)
    iota16 = lax.iota(jnp.int32, 16)

    def _zero(i, _):
        pres[pl.ds(i * 16, 16)] = zero16
        return 0

    lax.fori_loop(0, PBINS // 16, _zero, 0)

    def _group(g, _):
        base = wid * spw + g * GROUP
        pltpu.sync_copy(chars_hbm.at[pl.ds(base, GROUP)], chars_v)
        pltpu.sync_copy(colors_hbm.at[pl.ds(base, GROUP)], colors_v)

        def _sample(jj, _):
            s = base + jj
            marker = jnp.full((16,), s + 1, jnp.int32)

            def _scat16(off):
                c = chars_v[jj, pl.ds(off, 16)]
                l = colors_v[jj, pl.ds(off, 16)]
                plsc.store_scatter(pres, [c * 16 + l], marker)

            def _scatter(i, _):
                for u in range(4):
                    _scat16(i * 64 + u * 16)
                return 0

            lax.fori_loop(0, HW // 64, _scatter, 0)
            for off in (1600, 1616, 1632, HW - 16):
                _scat16(off)

            for cc in range(5):
                outbuf[pl.ds(cc * 16, 16)] = sent16

            def _cond(carry):
                i, off = carry
                return jnp.logical_and(i < NCHUNK, off < MAX_LEN)

            def _compact(carry):
                i, off = carry
                m = pres[pl.ds(i * 16, 16)] == marker
                mi = m.astype(jnp.int32)
                pos = plsc.cumsum(mi)
                plsc.store_scatter(outbuf, [off + pos - 1], iota16 + i * 16, mask=m)
                return i + 1, off + jnp.sum(mi)

            lax.while_loop(_cond, _compact, (jnp.int32(0), jnp.int32(0)))
            pltpu.sync_copy(outbuf.at[pl.ds(0, MAX_LEN)], keys_hbm.at[s])
            return 0

        lax.fori_loop(0, GROUP, _sample, 0)
        return 0

    lax.fori_loop(0, spw // GROUP, _group, 0)


@functools.cache
def _make_sc_bag():
    return pl.kernel(
        _sc_bag_body,
        mesh=plsc.VectorSubcoreMesh(core_axis_name="c", subcore_axis_name="s"),
        compiler_params=pltpu.CompilerParams(
            needs_layout_passes=False, use_tc_tiling_on_sc=False),
        out_type=jax.ShapeDtypeStruct((B, MAX_LEN), jnp.int32),
        scratch_types=[
            pltpu.VMEM((GROUP, HW), jnp.int32),
            pltpu.VMEM((GROUP, HW), jnp.int32),
            pltpu.VMEM((PBINS,), jnp.int32),
            pltpu.VMEM((80,), jnp.int32),
        ],
    )


def _tc_body(keys_ref, ct_ref, lt_ref, wih_ref, whh_ref, bih_ref, bhh_ref,
             feat_ref, emb_ref, cbag_ref, lbag_ref):
    ct = ct_ref[...]
    lt = lt_ref[...]
    keys = keys_ref[...]
    valid = keys < NBINS

    cbag_ref[...] = jnp.where(valid, keys >> 4, PAD_CHAR)
    lbag_ref[...] = jnp.where(valid, keys & 15, PAD_COLOR)

    ci = jnp.where(valid, keys >> 4, CHAR_DIM)
    lj = jnp.where(valid, keys & 15, COLOR_DIM) + (CHAR_DIM + 1)
    iota_j = lax.broadcasted_iota(jnp.int32, (1, 114), 1)

    wih = wih_ref[...]
    whh = whh_ref[...]
    bias = bih_ref[...] + bhh_ref[...]

    z97x4 = jnp.zeros((CHAR_DIM + 1, 4), jnp.float32)
    z17x16 = jnp.zeros((COLOR_DIM + 1, 16), jnp.float32)
    te = jnp.concatenate([
        jnp.concatenate([ct, z97x4], axis=1),
        jnp.concatenate([z17x16, lt], axis=1)], axis=0)
    tw = lax.dot_general(te, wih, (((1,), (1,)), ((), ())),
                         preferred_element_type=jnp.float32)
    tt = jnp.concatenate([te, tw], axis=1)
    tt_hi = tt.astype(jnp.bfloat16)
    tt_lo = (tt - tt_hi.astype(jnp.float32)).astype(jnp.bfloat16)

    h = jnp.zeros((B, HIDDEN), jnp.float32)
    for t in range(MAX_LEN):
        oh = ((ci[:, t:t + 1] == iota_j) | (lj[:, t:t + 1] == iota_j)
              ).astype(jnp.bfloat16)
        z = (lax.dot_general(oh, tt_hi, (((1,), (0,)), ((), ())),
                             preferred_element_type=jnp.float32)
             + lax.dot_general(oh, tt_lo, (((1,), (0,)), ((), ())),
                               preferred_element_type=jnp.float32))
        emb_ref[:, t * 20:(t + 1) * 20] = z[:, :20]
        hn = jnp.tanh(
            z[:, 20:]
            + lax.dot_general(h, whh, (((1,), (1,)), ((), ())),
                              preferred_element_type=jnp.float32)
            + bias)
        h = jnp.where(valid[:, t:t + 1], hn, h)
    feat_ref[...] = h


def _tc_embed_rnn(keys, char_table, color_table, W_ih, W_hh, b_ih, b_hh):
    return pl.pallas_call(
        _tc_body,
        out_shape=(
            jax.ShapeDtypeStruct((B, HIDDEN), jnp.float32),
            jax.ShapeDtypeStruct((B, MAX_LEN * 20), jnp.float32),
            jax.ShapeDtypeStruct((B, MAX_LEN), jnp.int32),
            jax.ShapeDtypeStruct((B, MAX_LEN), jnp.int32),
        ),
    )(keys, char_table, color_table, W_ih, W_hh,
      b_ih.reshape(1, HIDDEN), b_hh.reshape(1, HIDDEN))


def kernel(glyph_chars, glyph_colors, char_table, color_table, W_ih, W_hh, b_ih, b_hh):
    chars2 = glyph_chars.reshape(B, HW).astype(jnp.int32)
    colors2 = glyph_colors.reshape(B, HW).astype(jnp.int32)

    keys = _make_sc_bag()(chars2, colors2)
    features, emb_flat, cbag, lbag = _tc_embed_rnn(
        keys, char_table, color_table, W_ih, W_hh, b_ih, b_hh)
    bag = jnp.stack([cbag, lbag], axis=-1)
    return features, emb_flat.reshape(B, MAX_LEN, 20), bag


# speedup vs baseline: 1.5615x; 1.3743x over previous
import functools

import jax
import jax.numpy as jnp
from jax import lax
from jax.experimental import pallas as pl
from jax.experimental.pallas import tpu as pltpu
from jax.experimental.pallas import tpu_sc as plsc

B = 1024
H, W = 21, 79
HW = H * W
GPAD = 1664
CHAR_DIM = 96
COLOR_DIM = 16
NBINS = CHAR_DIM * COLOR_DIM
PBINS = 1664
NCHUNK = NBINS // 16
SENT = 2047
MAX_LEN = 64
PAD_CHAR = 128
PAD_COLOR = COLOR_DIM
HIDDEN = 32
GROUP = 8


def _sc_bag_body(chars_hbm, colors_hbm, keys_hbm, chars_v, colors_v, pres, outbuf):
    info = plsc.get_sparse_core_info()
    nc = info.num_cores
    nw = nc * info.num_subcores
    spw = B // nw
    wid = lax.axis_index("s") * nc + lax.axis_index("c")

    zero16 = jnp.zeros((16,), jnp.int32)
    sent16 = jnp.full((16,), SENT, jnp.int32)
    iota16 = lax.iota(jnp.int32, 16)

    def _zero(i, _):
        pres[pl.ds(i * 16, 16)] = zero16
        return 0

    lax.fori_loop(0, PBINS // 16, _zero, 0)

    def _group(g, _):
        base = wid * spw + g * GROUP
        pltpu.sync_copy(chars_hbm.at[pl.ds(base, GROUP)], chars_v)
        pltpu.sync_copy(colors_hbm.at[pl.ds(base, GROUP)], colors_v)

        def _sample(jj, _):
            s = base + jj
            marker = jnp.full((16,), s + 1, jnp.int32)

            def _scat16(off):
                c = chars_v[jj, pl.ds(off, 16)]
                l = colors_v[jj, pl.ds(off, 16)]
                plsc.store_scatter(pres, [c * 16 + l], marker)

            def _scatter(i, _):
                for u in range(4):
                    _scat16(i * 64 + u * 16)
                return 0

            lax.fori_loop(0, HW // 64, _scatter, 0)
            for off in (1600, 1616, 1632, HW - 16):
                _scat16(off)

            for cc in range(5):
                outbuf[pl.ds(cc * 16, 16)] = sent16

            def _cond(carry):
                i, off = carry
                return jnp.logical_and(i < NCHUNK, off < MAX_LEN)

            def _compact(carry):
                i, off = carry
                m = pres[pl.ds(i * 16, 16)] == marker
                mi = m.astype(jnp.int32)
                pos = plsc.cumsum(mi)
                plsc.store_scatter(outbuf, [off + pos - 1], iota16 + i * 16, mask=m)
                return i + 1, off + jnp.sum(mi)

            lax.while_loop(_cond, _compact, (jnp.int32(0), jnp.int32(0)))
            pltpu.sync_copy(outbuf.at[pl.ds(0, MAX_LEN)], keys_hbm.at[s])
            return 0

        lax.fori_loop(0, GROUP, _sample, 0)
        return 0

    lax.fori_loop(0, spw // GROUP, _group, 0)


@functools.cache
def _make_sc_bag():
    return pl.kernel(
        _sc_bag_body,
        mesh=plsc.VectorSubcoreMesh(core_axis_name="c", subcore_axis_name="s"),
        compiler_params=pltpu.CompilerParams(
            needs_layout_passes=False, use_tc_tiling_on_sc=False),
        out_type=jax.ShapeDtypeStruct((B, MAX_LEN), jnp.int32),
        scratch_types=[
            pltpu.VMEM((GROUP, HW), jnp.int32),
            pltpu.VMEM((GROUP, HW), jnp.int32),
            pltpu.VMEM((PBINS,), jnp.int32),
            pltpu.VMEM((80,), jnp.int32),
        ],
    )


TPC = 4


def _tc_body(keysT_ref, ct_ref, lt_ref, wih_ref, whh_ref, bih_ref, bhh_ref,
             feat_ref, emb_ref, cbag_ref, lbag_ref):
    ct = ct_ref[...]
    lt = lt_ref[...]
    keysT = keysT_ref[...]
    valid = keysT < NBINS

    cbag_ref[...] = jnp.where(valid, keysT >> 4, PAD_CHAR)
    lbag_ref[...] = jnp.where(valid, keysT & 15, PAD_COLOR)

    ci = jnp.where(valid, keysT >> 4, CHAR_DIM)
    lj = jnp.where(valid, keysT & 15, COLOR_DIM) + (CHAR_DIM + 1)
    iota_j = lax.broadcasted_iota(jnp.int32, (114, 1), 0)

    wih = wih_ref[...]
    whh = whh_ref[...]
    biasT = bih_ref[...] + bhh_ref[...]

    z97x4 = jnp.zeros((CHAR_DIM + 1, 4), jnp.float32)
    z17x16 = jnp.zeros((COLOR_DIM + 1, 16), jnp.float32)
    te = jnp.concatenate([
        jnp.concatenate([ct, z97x4], axis=1),
        jnp.concatenate([z17x16, lt], axis=1)], axis=0)
    tw = lax.dot_general(te, wih, (((1,), (1,)), ((), ())),
                         preferred_element_type=jnp.float32)
    tt = jnp.concatenate([te, tw], axis=1)
    eye52 = (lax.broadcasted_iota(jnp.int32, (52, 1), 0)
             == lax.broadcasted_iota(jnp.int32, (1, 52), 1)).astype(jnp.float32)
    ttT = lax.dot_general(eye52, tt, (((1,), (1,)), ((), ())),
                          preferred_element_type=jnp.float32)
    ttT_hi = ttT.astype(jnp.bfloat16)
    ttT_lo = (ttT - ttT_hi.astype(jnp.float32)).astype(jnp.bfloat16)

    hT = jnp.zeros((HIDDEN, B), jnp.float32)
    for r in range(MAX_LEN // TPC):
        c0 = r * TPC * B
        ohT = ((ci[r:r + 1, :] == iota_j) | (lj[r:r + 1, :] == iota_j)
               ).astype(jnp.bfloat16)
        zT = (lax.dot_general(ttT_hi, ohT, (((1,), (0,)), ((), ())),
                              preferred_element_type=jnp.float32)
              + lax.dot_general(ttT_lo, ohT, (((1,), (0,)), ((), ())),
                                preferred_element_type=jnp.float32))
        emb_ref[:, c0:c0 + TPC * B] = zT[:20, :]
        zrnn = zT[20:, :]
        for w in range(TPC):
            pre = (zrnn[:, w * B:(w + 1) * B]
                   + lax.dot_general(whh, hT, (((1,), (0,)), ((), ())),
                                     preferred_element_type=jnp.float32)
                   + biasT)
            vrow = valid[r:r + 1, w * B:(w + 1) * B]
            hT = jnp.where(vrow, jnp.tanh(pre), hT)
    feat_ref[...] = hT


def _tc_embed_rnn(keysT, char_table, color_table, W_ih, W_hh, b_ih, b_hh):
    return pl.pallas_call(
        _tc_body,
        out_shape=(
            jax.ShapeDtypeStruct((HIDDEN, B), jnp.float32),
            jax.ShapeDtypeStruct((20, MAX_LEN * B), jnp.float32),
            jax.ShapeDtypeStruct((MAX_LEN // TPC, TPC * B), jnp.int32),
            jax.ShapeDtypeStruct((MAX_LEN // TPC, TPC * B), jnp.int32),
        ),
    )(keysT.reshape(MAX_LEN // TPC, TPC * B), char_table, color_table,
      W_ih, W_hh, b_ih.reshape(HIDDEN, 1), b_hh.reshape(HIDDEN, 1))


def kernel(glyph_chars, glyph_colors, char_table, color_table, W_ih, W_hh, b_ih, b_hh):
    chars2 = glyph_chars.reshape(B, HW).astype(jnp.int32)
    colors2 = glyph_colors.reshape(B, HW).astype(jnp.int32)

    keys = _make_sc_bag()(chars2, colors2)
    keysT = keys.T
    featT, embT, cbagT, lbagT = _tc_embed_rnn(
        keysT, char_table, color_table, W_ih, W_hh, b_ih, b_hh)
    features = featT.T
    emb = embT.reshape(20, MAX_LEN, B).transpose(2, 1, 0)
    bag = jnp.stack([cbagT.reshape(MAX_LEN, B).T,
                     lbagT.reshape(MAX_LEN, B).T], axis=-1)
    return features, emb, bag


# speedup vs baseline: 1.7977x; 1.1512x over previous
import functools

import jax
import jax.numpy as jnp
from jax import lax
from jax.experimental import pallas as pl
from jax.experimental.pallas import tpu as pltpu
from jax.experimental.pallas import tpu_sc as plsc

B = 1024
H, W = 21, 79
HW = H * W
GPAD = 1664
CHAR_DIM = 96
COLOR_DIM = 16
NBINS = CHAR_DIM * COLOR_DIM
PBINS = 1664
NCHUNK = NBINS // 16
SENT = 2047
MAX_LEN = 64
PAD_CHAR = 128
PAD_COLOR = COLOR_DIM
HIDDEN = 32
GROUP = 8


def _sc_bag_body(glyphs_hbm, keys_hbm, glyphs_v, pres, outbuf):
    info = plsc.get_sparse_core_info()
    nc = info.num_cores
    nw = nc * info.num_subcores
    spw = B // nw
    wid = lax.axis_index("s") * nc + lax.axis_index("c")

    zero16 = jnp.zeros((16,), jnp.int32)
    sent16 = jnp.full((16,), SENT, jnp.int32)
    iota16 = lax.iota(jnp.int32, 16)

    def _zero(i, _):
        pres[pl.ds(i * 16, 16)] = zero16
        return 0

    lax.fori_loop(0, PBINS // 16, _zero, 0)

    def _group(g, _):
        base = wid * spw + g * GROUP
        pltpu.sync_copy(glyphs_hbm.at[pl.ds(base, GROUP)], glyphs_v)

        def _sample(jj, _):
            s = base + jj
            marker = jnp.full((16,), s + 1, jnp.int32)

            def _scat16(off):
                plsc.store_scatter(pres, [glyphs_v[jj, pl.ds(off, 16)]], marker)

            def _scatter(i, _):
                for u in range(4):
                    _scat16(i * 64 + u * 16)
                return 0

            lax.fori_loop(0, HW // 64, _scatter, 0)
            for off in (1600, 1616, 1632, HW - 16):
                _scat16(off)

            for cc in range(5):
                outbuf[pl.ds(cc * 16, 16)] = sent16

            def _cond(carry):
                i, off = carry
                return jnp.logical_and(i < NCHUNK, off < MAX_LEN)

            def _compact(carry):
                i, off = carry
                m = pres[pl.ds(i * 16, 16)] == marker
                mi = m.astype(jnp.int32)
                pos = plsc.cumsum(mi)
                plsc.store_scatter(outbuf, [off + pos - 1], iota16 + i * 16, mask=m)
                return i + 1, off + jnp.sum(mi)

            lax.while_loop(_cond, _compact, (jnp.int32(0), jnp.int32(0)))
            pltpu.sync_copy(outbuf.at[pl.ds(0, MAX_LEN)], keys_hbm.at[s])
            return 0

        lax.fori_loop(0, GROUP, _sample, 0)
        return 0

    lax.fori_loop(0, spw // GROUP, _group, 0)


@functools.cache
def _make_sc_bag():
    return pl.kernel(
        _sc_bag_body,
        mesh=plsc.VectorSubcoreMesh(core_axis_name="c", subcore_axis_name="s"),
        compiler_params=pltpu.CompilerParams(
            needs_layout_passes=False, use_tc_tiling_on_sc=False),
        out_type=jax.ShapeDtypeStruct((B, MAX_LEN), jnp.int32),
        scratch_types=[
            pltpu.VMEM((GROUP, HW), jnp.int32),
            pltpu.VMEM((PBINS,), jnp.int32),
            pltpu.VMEM((80,), jnp.int32),
        ],
    )


TPC = 4


def _tc_body(keysT_ref, ct_ref, lt_ref, wih_ref, whh_ref, bih_ref, bhh_ref,
             feat_ref, emb_ref, cbag_ref, lbag_ref):
    ct = ct_ref[...]
    lt = lt_ref[...]
    keysT = keysT_ref[...]
    valid = keysT < NBINS

    cbag_ref[...] = jnp.where(valid, keysT >> 4, PAD_CHAR)
    lbag_ref[...] = jnp.where(valid, keysT & 15, PAD_COLOR)

    ci = jnp.where(valid, keysT >> 4, CHAR_DIM)
    lj = jnp.where(valid, keysT & 15, COLOR_DIM) + (CHAR_DIM + 1)
    iota_j = lax.broadcasted_iota(jnp.int32, (114, 1), 0)

    wih = wih_ref[...]
    whh = whh_ref[...]
    biasT = bih_ref[...] + bhh_ref[...]

    z97x4 = jnp.zeros((CHAR_DIM + 1, 4), jnp.float32)
    z17x16 = jnp.zeros((COLOR_DIM + 1, 16), jnp.float32)
    te = jnp.concatenate([
        jnp.concatenate([ct, z97x4], axis=1),
        jnp.concatenate([z17x16, lt], axis=1)], axis=0)
    tw = lax.dot_general(te, wih, (((1,), (1,)), ((), ())),
                         preferred_element_type=jnp.float32)
    tt = jnp.concatenate([te, tw], axis=1)
    eye52 = (lax.broadcasted_iota(jnp.int32, (52, 1), 0)
             == lax.broadcasted_iota(jnp.int32, (1, 52), 1)).astype(jnp.float32)
    ttT = lax.dot_general(eye52, tt, (((1,), (1,)), ((), ())),
                          preferred_element_type=jnp.float32)
    ttT_hi = ttT.astype(jnp.bfloat16)
    ttT_lo = (ttT - ttT_hi.astype(jnp.float32)).astype(jnp.bfloat16)

    hT = jnp.zeros((HIDDEN, B), jnp.float32)
    for r in range(MAX_LEN // TPC):
        c0 = r * TPC * B
        ohT = ((ci[r:r + 1, :] == iota_j) | (lj[r:r + 1, :] == iota_j)
               ).astype(jnp.bfloat16)
        zT = (lax.dot_general(ttT_hi, ohT, (((1,), (0,)), ((), ())),
                              preferred_element_type=jnp.float32)
              + lax.dot_general(ttT_lo, ohT, (((1,), (0,)), ((), ())),
                                preferred_element_type=jnp.float32))
        emb_ref[:, c0:c0 + TPC * B] = zT[:20, :]
        zrnn = zT[20:, :]
        for w in range(TPC):
            pre = (zrnn[:, w * B:(w + 1) * B]
                   + lax.dot_general(whh, hT, (((1,), (0,)), ((), ())),
                                     preferred_element_type=jnp.float32)
                   + biasT)
            vrow = valid[r:r + 1, w * B:(w + 1) * B]
            hT = jnp.where(vrow, jnp.tanh(pre), hT)
    feat_ref[...] = hT


def _tc_embed_rnn(keysT, char_table, color_table, W_ih, W_hh, b_ih, b_hh):
    return pl.pallas_call(
        _tc_body,
        out_shape=(
            jax.ShapeDtypeStruct((HIDDEN, B), jnp.float32),
            jax.ShapeDtypeStruct((20, MAX_LEN * B), jnp.float32),
            jax.ShapeDtypeStruct((MAX_LEN // TPC, TPC * B), jnp.int32),
            jax.ShapeDtypeStruct((MAX_LEN // TPC, TPC * B), jnp.int32),
        ),
    )(keysT.reshape(MAX_LEN // TPC, TPC * B), char_table, color_table,
      W_ih, W_hh, b_ih.reshape(HIDDEN, 1), b_hh.reshape(HIDDEN, 1))


def kernel(glyph_chars, glyph_colors, char_table, color_table, W_ih, W_hh, b_ih, b_hh):
    glyphs = (glyph_chars.reshape(B, HW) * 16
              + glyph_colors.reshape(B, HW)).astype(jnp.int32)

    keys = _make_sc_bag()(glyphs)
    keysT = keys.T
    featT, embT, cbagT, lbagT = _tc_embed_rnn(
        keysT, char_table, color_table, W_ih, W_hh, b_ih, b_hh)
    features = featT.T
    emb = embT.reshape(20, MAX_LEN, B).transpose(2, 1, 0)
    bag = jnp.stack([cbagT.reshape(MAX_LEN, B).T,
                     lbagT.reshape(MAX_LEN, B).T], axis=-1)
    return features, emb, bag


# speedup vs baseline: 1.8739x; 1.0424x over previous
import functools

import jax
import jax.numpy as jnp
from jax import lax
from jax.experimental import pallas as pl
from jax.experimental.pallas import tpu as pltpu
from jax.experimental.pallas import tpu_sc as plsc

B = 1024
H, W = 21, 79
HW = H * W
GPAD = 1664
CHAR_DIM = 96
COLOR_DIM = 16
NBINS = CHAR_DIM * COLOR_DIM
PBINS = 1664
NCHUNK = NBINS // 16
SENT = 2047
MAX_LEN = 64
PAD_CHAR = 128
PAD_COLOR = COLOR_DIM
HIDDEN = 32
GROUP = 8


HB = B // 2


def _sc_bag_body_for(half):
  def _sc_bag_body(glyphs_hbm, keys_hbm, glyphs_v, pres, outbuf):
    info = plsc.get_sparse_core_info()
    nc = info.num_cores
    nw = nc * info.num_subcores
    spw = HB // nw
    wid = lax.axis_index("s") * nc + lax.axis_index("c")

    zero16 = jnp.zeros((16,), jnp.int32)
    sent16 = jnp.full((16,), SENT, jnp.int32)
    iota16 = lax.iota(jnp.int32, 16)

    def _zero(i, _):
        pres[pl.ds(i * 16, 16)] = zero16
        return 0

    lax.fori_loop(0, PBINS // 16, _zero, 0)

    def _group(g, _):
        lbase = wid * spw + g * GROUP
        base = half * HB + lbase
        pltpu.sync_copy(glyphs_hbm.at[pl.ds(base, GROUP)], glyphs_v)

        def _sample(jj, _):
            s = base + jj
            marker = jnp.full((16,), s + 1, jnp.int32)

            def _scat16(off):
                plsc.store_scatter(pres, [glyphs_v[jj, pl.ds(off, 16)]], marker)

            def _scatter(i, _):
                for u in range(4):
                    _scat16(i * 64 + u * 16)
                return 0

            lax.fori_loop(0, HW // 64, _scatter, 0)
            for off in (1600, 1616, 1632, HW - 16):
                _scat16(off)

            for cc in range(5):
                outbuf[pl.ds(cc * 16, 16)] = sent16

            def _cond(carry):
                i, off = carry
                return jnp.logical_and(i < NCHUNK, off < MAX_LEN)

            def _compact(carry):
                i, off = carry
                m = pres[pl.ds(i * 16, 16)] == marker
                mi = m.astype(jnp.int32)
                pos = plsc.cumsum(mi)
                plsc.store_scatter(outbuf, [off + pos - 1], iota16 + i * 16, mask=m)
                return i + 1, off + jnp.sum(mi)

            lax.while_loop(_cond, _compact, (jnp.int32(0), jnp.int32(0)))
            pltpu.sync_copy(outbuf.at[pl.ds(0, MAX_LEN)],
                            keys_hbm.at[lbase + jj])
            return 0

        lax.fori_loop(0, GROUP, _sample, 0)
        return 0

    lax.fori_loop(0, spw // GROUP, _group, 0)

  return _sc_bag_body


@functools.cache
def _make_sc_bag(half):
    return pl.kernel(
        _sc_bag_body_for(half),
        mesh=plsc.VectorSubcoreMesh(core_axis_name="c", subcore_axis_name="s"),
        compiler_params=pltpu.CompilerParams(
            needs_layout_passes=False, use_tc_tiling_on_sc=False),
        out_type=jax.ShapeDtypeStruct((HB, MAX_LEN), jnp.int32),
        scratch_types=[
            pltpu.VMEM((GROUP, HW), jnp.int32),
            pltpu.VMEM((PBINS,), jnp.int32),
            pltpu.VMEM((80,), jnp.int32),
        ],
    )


TPC = 4


def _tc_body(keysT_ref, ct_ref, lt_ref, wih_ref, whh_ref, bih_ref, bhh_ref,
             feat_ref, emb_ref, cbag_ref, lbag_ref):
    bh = HB
    ct = ct_ref[...]
    lt = lt_ref[...]
    keysT = keysT_ref[...]
    valid = keysT < NBINS

    cbag_ref[...] = jnp.where(valid, keysT >> 4, PAD_CHAR)
    lbag_ref[...] = jnp.where(valid, keysT & 15, PAD_COLOR)

    ci = jnp.where(valid, keysT >> 4, CHAR_DIM)
    lj = jnp.where(valid, keysT & 15, COLOR_DIM) + (CHAR_DIM + 1)
    iota_j = lax.broadcasted_iota(jnp.int32, (114, 1), 0)

    wih = wih_ref[...]
    whh = whh_ref[...]
    biasT = bih_ref[...] + bhh_ref[...]

    z97x4 = jnp.zeros((CHAR_DIM + 1, 4), jnp.float32)
    z17x16 = jnp.zeros((COLOR_DIM + 1, 16), jnp.float32)
    te = jnp.concatenate([
        jnp.concatenate([ct, z97x4], axis=1),
        jnp.concatenate([z17x16, lt], axis=1)], axis=0)
    tw = lax.dot_general(te, wih, (((1,), (1,)), ((), ())),
                         preferred_element_type=jnp.float32)
    tt = jnp.concatenate([te, tw], axis=1)
    eye52 = (lax.broadcasted_iota(jnp.int32, (52, 1), 0)
             == lax.broadcasted_iota(jnp.int32, (1, 52), 1)).astype(jnp.float32)
    ttT = lax.dot_general(eye52, tt, (((1,), (1,)), ((), ())),
                          preferred_element_type=jnp.float32)
    ttT_hi = ttT.astype(jnp.bfloat16)
    ttT_lo = (ttT - ttT_hi.astype(jnp.float32)).astype(jnp.bfloat16)

    hT = jnp.zeros((HIDDEN, bh), jnp.float32)
    for r in range(MAX_LEN // TPC):
        c0 = r * TPC * bh
        ohT = ((ci[r:r + 1, :] == iota_j) | (lj[r:r + 1, :] == iota_j)
               ).astype(jnp.bfloat16)
        zT = (lax.dot_general(ttT_hi, ohT, (((1,), (0,)), ((), ())),
                              preferred_element_type=jnp.float32)
              + lax.dot_general(ttT_lo, ohT, (((1,), (0,)), ((), ())),
                                preferred_element_type=jnp.float32))
        emb_ref[:, c0:c0 + TPC * bh] = zT[:20, :]
        zrnn = zT[20:, :]
        for w in range(TPC):
            pre = (zrnn[:, w * bh:(w + 1) * bh]
                   + lax.dot_general(whh, hT, (((1,), (0,)), ((), ())),
                                     preferred_element_type=jnp.float32)
                   + biasT)
            vrow = valid[r:r + 1, w * bh:(w + 1) * bh]
            hT = jnp.where(vrow, jnp.tanh(pre), hT)
    feat_ref[...] = hT


def _tc_embed_rnn(keysT, char_table, color_table, W_ih, W_hh, b_ih, b_hh):
    return pl.pallas_call(
        _tc_body,
        out_shape=(
            jax.ShapeDtypeStruct((HIDDEN, HB), jnp.float32),
            jax.ShapeDtypeStruct((20, MAX_LEN * HB), jnp.float32),
            jax.ShapeDtypeStruct((MAX_LEN // TPC, TPC * HB), jnp.int32),
            jax.ShapeDtypeStruct((MAX_LEN // TPC, TPC * HB), jnp.int32),
        ),
    )(keysT.reshape(MAX_LEN // TPC, TPC * HB), char_table, color_table,
      W_ih, W_hh, b_ih.reshape(HIDDEN, 1), b_hh.reshape(HIDDEN, 1))


def kernel(glyph_chars, glyph_colors, char_table, color_table, W_ih, W_hh, b_ih, b_hh):
    glyphs = (glyph_chars.reshape(B, HW) * 16
              + glyph_colors.reshape(B, HW)).astype(jnp.int32)

    halves = []
    for half in range(2):
        keys_h = _make_sc_bag(half)(glyphs)
        halves.append(_tc_embed_rnn(
            keys_h.T, char_table, color_table, W_ih, W_hh, b_ih, b_hh))
    features = jnp.concatenate([fT.T for fT, _, _, _ in halves], axis=0)
    emb = jnp.concatenate(
        [eT.reshape(20, MAX_LEN, HB).transpose(2, 1, 0)
         for _, eT, _, _ in halves], axis=0)
    bag = jnp.stack([
        jnp.concatenate([cT.reshape(MAX_LEN, HB).T
                         for _, _, cT, _ in halves], axis=0),
        jnp.concatenate([lT.reshape(MAX_LEN, HB).T
                         for _, _, _, lT in halves], axis=0)], axis=-1)
    return features, emb, bag
